# baseline (speedup 1.0000x reference)
import functools

import jax
import jax.numpy as jnp
from jax import lax
from jax.experimental import pallas as pl
from jax.experimental.pallas import tpu as pltpu
from jax.experimental.pallas import tpu_sc as plsc

N = 10000
E = 160000
LAT = 128
H1 = 256
H2 = 128

_NC = 2
_NS = 16
_NW = _NC * _NS

f32 = jnp.float32
_PHI = jax.lax.Precision.HIGHEST


def _tc_nodes(x, W, b):
    def body(x_ref, w_ref, b_ref, o_ref):
        o_ref[...] = jnp.dot(x_ref[...], w_ref[...],
                             preferred_element_type=f32, precision=_PHI) + b_ref[...]

    return pl.pallas_call(
        body,
        grid=(10,),
        in_specs=[
            pl.BlockSpec((1000, 128), lambda i: (i, 0)),
            pl.BlockSpec((128, 128), lambda i: (0, 0)),
            pl.BlockSpec((1, 128), lambda i: (0, 0)),
        ],
        out_specs=pl.BlockSpec((1000, 128), lambda i: (i, 0)),
        out_shape=jax.ShapeDtypeStruct((N, 128), f32),
    )(x, W, b)


_G_PER_W = (2 * E) // _NW
_G_CH = 128
_G_NFULL = _G_PER_W // _G_CH
_G_TAIL = _G_PER_W - _G_NFULL * _G_CH


def _sc_gather(table, idx):
    mesh = plsc.VectorSubcoreMesh(core_axis_name="c", subcore_axis_name="s")

    @functools.partial(
        pl.kernel,
        out_type=jax.ShapeDtypeStruct((2 * E, 128), f32),
        mesh=mesh,
        scratch_types=[
            pltpu.VMEM((_G_PER_W,), jnp.int32),
            pltpu.VMEM((_G_CH, 128), f32),
            pltpu.VMEM((_G_CH, 128), f32),
            pltpu.SemaphoreType.DMA,
            pltpu.SemaphoreType.DMA,
        ],
    )
    def k(table_hbm, idx_hbm, out_hbm, idx_v, rows0, rows1, sem0, sem1):
        wid = lax.axis_index("s") * _NC + lax.axis_index("c")
        base = wid * _G_PER_W
        pltpu.sync_copy(idx_hbm.at[pl.ds(base, _G_PER_W)], idx_v)

        rows = (rows0, rows1)
        sems = (sem0, sem1)

        def start(jj, b):
            pltpu.async_copy(
                table_hbm.at[idx_v.at[pl.ds(jj * _G_CH, _G_CH)]],
                rows[b], sems[b])

        def finish(b):
            pltpu.make_async_copy(
                table_hbm.at[idx_v.at[pl.ds(0, _G_CH)]],
                rows[b], sems[b]).wait()

        def store(jj, b):
            pltpu.sync_copy(rows[b],
                            out_hbm.at[pl.ds(base + jj * _G_CH, _G_CH)])

        start(0, 0)

        @pl.loop(0, _G_NFULL, step=2)
        def _(j):
            finish(0)
            start(j + 1, 1)
            store(j, 0)
            finish(1)

            @pl.when(j + 2 < _G_NFULL)
            def _():
                start(j + 2, 0)

            store(j + 1, 1)

        toff = base + _G_NFULL * _G_CH
        pltpu.async_copy(
            table_hbm.at[idx_v.at[pl.ds(_G_NFULL * _G_CH, _G_TAIL)]],
            rows0.at[pl.ds(0, _G_TAIL)], sem0).wait()
        pltpu.sync_copy(rows0.at[pl.ds(0, _G_TAIL)],
                        out_hbm.at[pl.ds(toff, _G_TAIL)])

    return k(table, idx)


_BE = 2000


def _tc_edge(sr, ea, Aee, Ws, Wr, We2, cee, be2):
    def body(snt_ref, rcv_ref, ea_ref, aee_ref, ws_ref, wr_ref, we2_ref,
             cee_ref, be2_ref, o_ref):
        h = jnp.dot(ea_ref[...], aee_ref[...], preferred_element_type=f32, precision=_PHI)
        h = h + jnp.dot(snt_ref[...], ws_ref[...], preferred_element_type=f32, precision=_PHI)
        h = h + jnp.dot(rcv_ref[...], wr_ref[...], preferred_element_type=f32, precision=_PHI)
        h = jnp.maximum(h + cee_ref[...], 0.0)
        o_ref[...] = jnp.dot(h, we2_ref[...],
                             preferred_element_type=f32, precision=_PHI) + be2_ref[...]

    nblk = E // _BE
    return pl.pallas_call(
        body,
        grid=(nblk,),
        in_specs=[
            pl.BlockSpec((_BE, 128), lambda i: (i, 0)),
            pl.BlockSpec((_BE, 128), lambda i: (i + nblk, 0)),
            pl.BlockSpec((_BE, 16), lambda i: (i, 0)),
            pl.BlockSpec((16, 256), lambda i: (0, 0)),
            pl.BlockSpec((128, 256), lambda i: (0, 0)),
            pl.BlockSpec((128, 256), lambda i: (0, 0)),
            pl.BlockSpec((256, 128), lambda i: (0, 0)),
            pl.BlockSpec((1, 256), lambda i: (0, 0)),
            pl.BlockSpec((1, 128), lambda i: (0, 0)),
        ],
        out_specs=pl.BlockSpec((_BE, 128), lambda i: (i, 0)),
        out_shape=jax.ShapeDtypeStruct((E, 128), f32),
    )(sr, sr, ea, Aee, Ws, Wr, We2, cee, be2)


_S_PER_T = E // _NS
_S_CH = 64
_S_NFULL = _S_PER_T // _S_CH
_S_TAIL = _S_PER_T - _S_NFULL * _S_CH
_S_OCH = 200
_S_ONCH = N // _S_OCH
_S_ZCH = 40
_S_ZNCH = N // _S_ZCH


def _sc_scatter(en, sr_idx):
    mesh = plsc.VectorSubcoreMesh(core_axis_name="c", subcore_axis_name="s")

    @functools.partial(
        pl.kernel,
        out_type=jax.ShapeDtypeStruct((2, N, 128), f32),
        mesh=mesh,
        scratch_types=[
            pltpu.VMEM_SHARED((N, 128), f32),
            pltpu.VMEM((_S_CH, 128), f32),
            pltpu.VMEM((_S_CH, 128), f32),
            pltpu.VMEM((_S_PER_T,), jnp.int32),
            pltpu.VMEM((_S_CH,), jnp.int32),
            pltpu.VMEM((_S_CH,), jnp.int32),
            pltpu.VMEM((_S_TAIL,), jnp.int32),
            pltpu.SemaphoreType.DMA,
            pltpu.SemaphoreType.DMA,
        ],
    )
    def k(en_hbm, sr_hbm, out_hbm, acc_sh, rows0, rows1, slab_v,
          idx0, idx1, idxt_v, sem0, sem1):
        c = lax.axis_index("c")
        s = lax.axis_index("s")

        @pl.loop(0, _S_ZCH)
        def _(i):
            for jj in range(8):
                rows0[i, pl.ds(jj * 16, 16)] = jnp.zeros((16,), f32)

        nzch = jnp.where(s < _S_ZNCH - (_S_ZNCH // _NS) * _NS,
                         _S_ZNCH // _NS + 1, _S_ZNCH // _NS)

        @pl.loop(0, nzch)
        def _(t):
            ch = s + t * _NS
            pltpu.sync_copy(rows0.at[pl.ds(0, _S_ZCH)],
                            acc_sh.at[pl.ds(ch * _S_ZCH, _S_ZCH)])

        plsc.subcore_barrier()

        ebase = s * _S_PER_T
        pltpu.sync_copy(sr_hbm.at[pl.ds(c * E + ebase, _S_PER_T)], slab_v)

        rows = (rows0, rows1)
        idxc = (idx0, idx1)
        sems = (sem0, sem1)

        def load(jj, b):
            pltpu.async_copy(en_hbm.at[pl.ds(ebase + jj * _S_CH, _S_CH)],
                             rows[b], sems[b])

        def loaded(b):
            pltpu.make_async_copy(en_hbm.at[pl.ds(ebase, _S_CH)],
                                  rows[b], sems[b]).wait()

        def scat(jj, b):
            for v in range(_S_CH // 16):
                idxc[b][pl.ds(v * 16, 16)] = (
                    slab_v[pl.ds(jj * _S_CH + v * 16, 16)])
            pltpu.sync_copy(rows[b], acc_sh.at[idxc[b]], add=True)

        load(0, 0)

        @pl.loop(0, _S_NFULL, step=2)
        def _(j):
            loaded(0)
            load(j + 1, 1)
            scat(j, 0)
            loaded(1)

            @pl.when(j + 2 < _S_NFULL)
            def _():
                load(j + 2, 0)

            scat(j + 1, 1)

        toff = ebase + _S_NFULL * _S_CH
        pltpu.sync_copy(en_hbm.at[pl.ds(toff, _S_TAIL)],
                        rows0.at[pl.ds(0, _S_TAIL)])
        pltpu.sync_copy(sr_hbm.at[pl.ds(c * E + toff, _S_TAIL)], idxt_v)
        pltpu.sync_copy(rows0.at[pl.ds(0, _S_TAIL)], acc_sh.at[idxt_v],
                        add=True)

        plsc.subcore_barrier()

        noch = jnp.where(s < _S_ONCH - (_S_ONCH // _NS) * _NS,
                         _S_ONCH // _NS + 1, _S_ONCH // _NS)

        @pl.loop(0, noch)
        def _(t):
            rr = (s + t * _NS) * _S_OCH
            pltpu.sync_copy(acc_sh.at[pl.ds(rr, _S_OCH)],
                            out_hbm.at[c, pl.ds(rr, _S_OCH)])

    return k(en, sr_idx)


_BN = 1000


def _tc_node_global(nodes, sent, recv, Wn1n, Wn1s, Wn1r, bn1, Wn2, bn2,
                    Wg1n, Wg1e, bg1, Wg2, bg2, wg3row, bg3):
    G = N // _BN

    def body(nd, sa, ra, wn1n, wn1s, wn1r, bn1r, wn2, bn2r,
             wg1n, wg1e, bg1r, wg2, bg2r, wg3r, bg3r, o_ref, acc_n, acc_e):
        i = pl.program_id(0)

        @pl.when(i == 0)
        def _():
            acc_n[...] = jnp.zeros_like(acc_n)
            acc_e[...] = jnp.zeros_like(acc_e)

        h = jnp.dot(nd[...], wn1n[...], preferred_element_type=f32, precision=_PHI)
        h = h + jnp.dot(sa[...], wn1s[...], preferred_element_type=f32, precision=_PHI)
        h = h + jnp.dot(ra[...], wn1r[...], preferred_element_type=f32, precision=_PHI)
        h = jnp.maximum(h + bn1r[...], 0.0)
        nn = jnp.dot(h, wn2[...], preferred_element_type=f32, precision=_PHI) + bn2r[...]
        acc_n[...] += nn.reshape(_BN // 8, 8, 128).sum(0)
        acc_e[...] += sa[...].reshape(_BN // 8, 8, 128).sum(0)

        @pl.when(i == G - 1)
        def _():
            na = jnp.sum(acc_n[...], axis=0, keepdims=True)
            eg = jnp.sum(acc_e[...], axis=0, keepdims=True)
            hg = jnp.dot(na, wg1n[...], preferred_element_type=f32, precision=_PHI)
            hg = hg + jnp.dot(eg, wg1e[...], preferred_element_type=f32, precision=_PHI)
            hg = jnp.maximum(hg + bg1r[...], 0.0)
            hg2 = jnp.maximum(
                jnp.dot(hg, wg2[...], preferred_element_type=f32, precision=_PHI) + bg2r[...],
                0.0)
            o_ref[...] = (jnp.sum(hg2 * wg3r[...], axis=1, keepdims=True)
                          + bg3r[...])

    return pl.pallas_call(
        body,
        grid=(G,),
        in_specs=[
            pl.BlockSpec((_BN, 128), lambda i: (i, 0)),
            pl.BlockSpec((_BN, 128), lambda i: (i, 0)),
            pl.BlockSpec((_BN, 128), lambda i: (i, 0)),
            pl.BlockSpec((128, 256), lambda i: (0, 0)),
            pl.BlockSpec((128, 256), lambda i: (0, 0)),
            pl.BlockSpec((128, 256), lambda i: (0, 0)),
            pl.BlockSpec((1, 256), lambda i: (0, 0)),
            pl.BlockSpec((256, 128), lambda i: (0, 0)),
            pl.BlockSpec((1, 128), lambda i: (0, 0)),
            pl.BlockSpec((128, 256), lambda i: (0, 0)),
            pl.BlockSpec((128, 256), lambda i: (0, 0)),
            pl.BlockSpec((1, 256), lambda i: (0, 0)),
            pl.BlockSpec((256, 128), lambda i: (0, 0)),
            pl.BlockSpec((1, 128), lambda i: (0, 0)),
            pl.BlockSpec((1, 128), lambda i: (0, 0)),
            pl.BlockSpec((1, 1), lambda i: (0, 0)),
        ],
        out_specs=pl.BlockSpec((1, 1), lambda i: (0, 0)),
        out_shape=jax.ShapeDtypeStruct((1, 1), f32),
        scratch_shapes=[pltpu.VMEM((8, 128), f32), pltpu.VMEM((8, 128), f32)],
    )(nodes, sent, recv, Wn1n, Wn1s, Wn1r, bn1, Wn2, bn2,
      Wg1n, Wg1e, bg1, Wg2, bg2, wg3row, bg3)


def kernel(x, edge_attr, senders, receivers, W_en, b_en, W_ee, b_ee,
           We1, be1, We2, be2, Wn1, bn1, Wn2, bn2,
           Wg1, bg1, Wg2, bg2, Wg3, bg3):
    Aee = W_ee @ We1[:128]
    cee = (b_ee @ We1[:128] + be1).reshape(1, H1)
    Wsr = We1[128:384]

    nodes = _tc_nodes(x, W_en, b_en.reshape(1, 128))

    sr_idx = jnp.concatenate([senders, receivers])
    sr = _sc_gather(nodes, sr_idx)

    en = _tc_edge(sr, edge_attr, Aee, Wsr[:128], Wsr[128:], We2, cee,
                  be2.reshape(1, 128))

    aggs = _sc_scatter(en, sr_idx)

    return _tc_node_global(
        nodes, aggs[0], aggs[1],
        Wn1[:128], Wn1[128:256], Wn1[256:384], bn1.reshape(1, H1),
        Wn2, bn2.reshape(1, 128),
        Wg1[:128], Wg1[128:256], bg1.reshape(1, H1),
        Wg2, bg2.reshape(1, 128), Wg3.reshape(1, 128), bg3.reshape(1, 1))


# speedup vs baseline: 2.1935x; 2.1935x over previous
import functools

import jax
import jax.numpy as jnp
from jax import lax
from jax.experimental import pallas as pl
from jax.experimental.pallas import tpu as pltpu
from jax.experimental.pallas import tpu_sc as plsc

N = 10000
E = 160000
LAT = 128
H1 = 256
H2 = 128

_NC = 2
_NS = 16
_NW = _NC * _NS

f32 = jnp.float32


def _tc_nodes(x, W, b):
    def body(x_ref, w_ref, b_ref, o_ref):
        o_ref[...] = jnp.dot(x_ref[...], w_ref[...],
                             preferred_element_type=f32) + b_ref[...]

    return pl.pallas_call(
        body,
        grid=(10,),
        in_specs=[
            pl.BlockSpec((1000, 128), lambda i: (i, 0)),
            pl.BlockSpec((128, 128), lambda i: (0, 0)),
            pl.BlockSpec((1, 128), lambda i: (0, 0)),
        ],
        out_specs=pl.BlockSpec((1000, 128), lambda i: (i, 0)),
        out_shape=jax.ShapeDtypeStruct((N, 128), f32),
    )(x, W, b)


_G_PER_W = (2 * E) // _NW
_G_CH = 128
_G_NFULL = _G_PER_W // _G_CH
_G_TAIL = _G_PER_W - _G_NFULL * _G_CH


def _sc_gather(table, idx):
    mesh = plsc.VectorSubcoreMesh(core_axis_name="c", subcore_axis_name="s")

    @functools.partial(
        pl.kernel,
        out_type=jax.ShapeDtypeStruct((2 * E, 128), f32),
        mesh=mesh,
        scratch_types=[
            pltpu.VMEM((_G_PER_W,), jnp.int32),
            pltpu.VMEM((_G_CH, 128), f32),
            pltpu.VMEM((_G_CH, 128), f32),
            pltpu.SemaphoreType.DMA,
            pltpu.SemaphoreType.DMA,
        ],
    )
    def k(table_hbm, idx_hbm, out_hbm, idx_v, rows0, rows1, sem0, sem1):
        wid = lax.axis_index("s") * _NC + lax.axis_index("c")
        base = wid * _G_PER_W
        pltpu.sync_copy(idx_hbm.at[pl.ds(base, _G_PER_W)], idx_v)

        rows = (rows0, rows1)
        sems = (sem0, sem1)

        def start(jj, b):
            pltpu.async_copy(
                table_hbm.at[idx_v.at[pl.ds(jj * _G_CH, _G_CH)]],
                rows[b], sems[b])

        def finish(b):
            pltpu.make_async_copy(
                table_hbm.at[idx_v.at[pl.ds(0, _G_CH)]],
                rows[b], sems[b]).wait()

        def store(jj, b):
            pltpu.sync_copy(rows[b],
                            out_hbm.at[pl.ds(base + jj * _G_CH, _G_CH)])

        start(0, 0)

        @pl.loop(0, _G_NFULL, step=2)
        def _(j):
            finish(0)
            start(j + 1, 1)
            store(j, 0)
            finish(1)

            @pl.when(j + 2 < _G_NFULL)
            def _():
                start(j + 2, 0)

            store(j + 1, 1)

        toff = base + _G_NFULL * _G_CH
        pltpu.async_copy(
            table_hbm.at[idx_v.at[pl.ds(_G_NFULL * _G_CH, _G_TAIL)]],
            rows0.at[pl.ds(0, _G_TAIL)], sem0).wait()
        pltpu.sync_copy(rows0.at[pl.ds(0, _G_TAIL)],
                        out_hbm.at[pl.ds(toff, _G_TAIL)])

    return k(table, idx)


_BE = 2000


def _tc_edge(sr, ea, W_ee, b_ee, We1e, Ws, Wr, We2, be1, be2):
    def body(snt_ref, rcv_ref, ea_ref, wee_ref, bee_ref, we1e_ref, ws_ref,
             wr_ref, we2_ref, be1_ref, be2_ref, o_ref):
        el = jnp.dot(ea_ref[...], wee_ref[...],
                     preferred_element_type=f32) + bee_ref[...]
        h = jnp.dot(el, we1e_ref[...], preferred_element_type=f32)
        h = h + jnp.dot(snt_ref[...], ws_ref[...], preferred_element_type=f32)
        h = h + jnp.dot(rcv_ref[...], wr_ref[...], preferred_element_type=f32)
        h = jnp.maximum(h + be1_ref[...], 0.0)
        o_ref[...] = jnp.dot(h, we2_ref[...],
                             preferred_element_type=f32) + be2_ref[...]

    nblk = E // _BE
    return pl.pallas_call(
        body,
        grid=(nblk,),
        in_specs=[
            pl.BlockSpec((_BE, 128), lambda i: (i, 0)),
            pl.BlockSpec((_BE, 128), lambda i: (i + nblk, 0)),
            pl.BlockSpec((_BE, 16), lambda i: (i, 0)),
            pl.BlockSpec((16, 128), lambda i: (0, 0)),
            pl.BlockSpec((1, 128), lambda i: (0, 0)),
            pl.BlockSpec((128, 256), lambda i: (0, 0)),
            pl.BlockSpec((128, 256), lambda i: (0, 0)),
            pl.BlockSpec((128, 256), lambda i: (0, 0)),
            pl.BlockSpec((256, 128), lambda i: (0, 0)),
            pl.BlockSpec((1, 256), lambda i: (0, 0)),
            pl.BlockSpec((1, 128), lambda i: (0, 0)),
        ],
        out_specs=pl.BlockSpec((_BE, 128), lambda i: (i, 0)),
        out_shape=jax.ShapeDtypeStruct((E, 128), f32),
    )(sr, sr, ea, W_ee, b_ee, We1e, Ws, Wr, We2, be1, be2)


_S_PER_T = E // _NS
_S_CH = 64
_S_NFULL = _S_PER_T // _S_CH
_S_TAIL = _S_PER_T - _S_NFULL * _S_CH
_S_OCH = 200
_S_ONCH = N // _S_OCH
_S_ZCH = 40
_S_ZNCH = N // _S_ZCH


def _sc_scatter(en, sr_idx):
    mesh = plsc.VectorSubcoreMesh(core_axis_name="c", subcore_axis_name="s")

    @functools.partial(
        pl.kernel,
        out_type=jax.ShapeDtypeStruct((2, N, 128), f32),
        mesh=mesh,
        scratch_types=[
            pltpu.VMEM_SHARED((N, 128), f32),
            pltpu.VMEM((_S_CH, 128), f32),
            pltpu.VMEM((_S_CH, 128), f32),
            pltpu.VMEM((_S_PER_T,), jnp.int32),
            pltpu.VMEM((_S_CH,), jnp.int32),
            pltpu.VMEM((_S_CH,), jnp.int32),
            pltpu.VMEM((_S_TAIL,), jnp.int32),
            pltpu.SemaphoreType.DMA,
            pltpu.SemaphoreType.DMA,
        ],
    )
    def k(en_hbm, sr_hbm, out_hbm, acc_sh, rows0, rows1, slab_v,
          idx0, idx1, idxt_v, sem0, sem1):
        c = lax.axis_index("c")
        s = lax.axis_index("s")

        @pl.loop(0, _S_ZCH)
        def _(i):
            for jj in range(8):
                rows0[i, pl.ds(jj * 16, 16)] = jnp.zeros((16,), f32)

        nzch = jnp.where(s < _S_ZNCH - (_S_ZNCH // _NS) * _NS,
                         _S_ZNCH // _NS + 1, _S_ZNCH // _NS)

        @pl.loop(0, nzch)
        def _(t):
            ch = s + t * _NS
            pltpu.sync_copy(rows0.at[pl.ds(0, _S_ZCH)],
                            acc_sh.at[pl.ds(ch * _S_ZCH, _S_ZCH)])

        plsc.subcore_barrier()

        ebase = s * _S_PER_T
        pltpu.sync_copy(sr_hbm.at[pl.ds(c * E + ebase, _S_PER_T)], slab_v)

        rows = (rows0, rows1)
        idxc = (idx0, idx1)
        sems = (sem0, sem1)

        def load(jj, b):
            pltpu.async_copy(en_hbm.at[pl.ds(ebase + jj * _S_CH, _S_CH)],
                             rows[b], sems[b])

        def loaded(b):
            pltpu.make_async_copy(en_hbm.at[pl.ds(ebase, _S_CH)],
                                  rows[b], sems[b]).wait()

        def scat(jj, b):
            for v in range(_S_CH // 16):
                idxc[b][pl.ds(v * 16, 16)] = (
                    slab_v[pl.ds(jj * _S_CH + v * 16, 16)])
            pltpu.sync_copy(rows[b], acc_sh.at[idxc[b]], add=True)

        load(0, 0)

        @pl.loop(0, _S_NFULL, step=2)
        def _(j):
            loaded(0)
            load(j + 1, 1)
            scat(j, 0)
            loaded(1)

            @pl.when(j + 2 < _S_NFULL)
            def _():
                load(j + 2, 0)

            scat(j + 1, 1)

        toff = ebase + _S_NFULL * _S_CH
        pltpu.sync_copy(en_hbm.at[pl.ds(toff, _S_TAIL)],
                        rows0.at[pl.ds(0, _S_TAIL)])
        pltpu.sync_copy(sr_hbm.at[pl.ds(c * E + toff, _S_TAIL)], idxt_v)
        pltpu.sync_copy(rows0.at[pl.ds(0, _S_TAIL)], acc_sh.at[idxt_v],
                        add=True)

        plsc.subcore_barrier()

        noch = jnp.where(s < _S_ONCH - (_S_ONCH // _NS) * _NS,
                         _S_ONCH // _NS + 1, _S_ONCH // _NS)

        @pl.loop(0, noch)
        def _(t):
            rr = (s + t * _NS) * _S_OCH
            pltpu.sync_copy(acc_sh.at[pl.ds(rr, _S_OCH)],
                            out_hbm.at[c, pl.ds(rr, _S_OCH)])

    return k(en, sr_idx)


_BN = 1000


def _tc_node_global(nodes, sent, recv, Wn1n, Wn1s, Wn1r, bn1, Wn2, bn2,
                    Wg1n, Wg1e, bg1, Wg2, bg2, wg3row, bg3):
    G = N // _BN

    def body(nd, sa, ra, wn1n, wn1s, wn1r, bn1r, wn2, bn2r,
             wg1n, wg1e, bg1r, wg2, bg2r, wg3r, bg3r, o_ref, acc_n, acc_e):
        i = pl.program_id(0)

        @pl.when(i == 0)
        def _():
            acc_n[...] = jnp.zeros_like(acc_n)
            acc_e[...] = jnp.zeros_like(acc_e)

        h = jnp.dot(nd[...], wn1n[...], preferred_element_type=f32)
        h = h + jnp.dot(sa[...], wn1s[...], preferred_element_type=f32)
        h = h + jnp.dot(ra[...], wn1r[...], preferred_element_type=f32)
        h = jnp.maximum(h + bn1r[...], 0.0)
        nn = jnp.dot(h, wn2[...], preferred_element_type=f32) + bn2r[...]
        acc_n[...] += nn.reshape(_BN // 8, 8, 128).sum(0)
        acc_e[...] += sa[...].reshape(_BN // 8, 8, 128).sum(0)

        @pl.when(i == G - 1)
        def _():
            na = jnp.sum(acc_n[...], axis=0, keepdims=True)
            eg = jnp.sum(acc_e[...], axis=0, keepdims=True)
            hg = jnp.dot(na, wg1n[...], preferred_element_type=f32)
            hg = hg + jnp.dot(eg, wg1e[...], preferred_element_type=f32)
            hg = jnp.maximum(hg + bg1r[...], 0.0)
            hg2 = jnp.maximum(
                jnp.dot(hg, wg2[...], preferred_element_type=f32) + bg2r[...],
                0.0)
            o_ref[...] = (jnp.sum(hg2 * wg3r[...], axis=1, keepdims=True)
                          + bg3r[...])

    return pl.pallas_call(
        body,
        grid=(G,),
        in_specs=[
            pl.BlockSpec((_BN, 128), lambda i: (i, 0)),
            pl.BlockSpec((_BN, 128), lambda i: (i, 0)),
            pl.BlockSpec((_BN, 128), lambda i: (i, 0)),
            pl.BlockSpec((128, 256), lambda i: (0, 0)),
            pl.BlockSpec((128, 256), lambda i: (0, 0)),
            pl.BlockSpec((128, 256), lambda i: (0, 0)),
            pl.BlockSpec((1, 256), lambda i: (0, 0)),
            pl.BlockSpec((256, 128), lambda i: (0, 0)),
            pl.BlockSpec((1, 128), lambda i: (0, 0)),
            pl.BlockSpec((128, 256), lambda i: (0, 0)),
            pl.BlockSpec((128, 256), lambda i: (0, 0)),
            pl.BlockSpec((1, 256), lambda i: (0, 0)),
            pl.BlockSpec((256, 128), lambda i: (0, 0)),
            pl.BlockSpec((1, 128), lambda i: (0, 0)),
            pl.BlockSpec((1, 128), lambda i: (0, 0)),
            pl.BlockSpec((1, 1), lambda i: (0, 0)),
        ],
        out_specs=pl.BlockSpec((1, 1), lambda i: (0, 0)),
        out_shape=jax.ShapeDtypeStruct((1, 1), f32),
        scratch_shapes=[pltpu.VMEM((8, 128), f32), pltpu.VMEM((8, 128), f32)],
    )(nodes, sent, recv, Wn1n, Wn1s, Wn1r, bn1, Wn2, bn2,
      Wg1n, Wg1e, bg1, Wg2, bg2, wg3row, bg3)


def kernel(x, edge_attr, senders, receivers, W_en, b_en, W_ee, b_ee,
           We1, be1, We2, be2, Wn1, bn1, Wn2, bn2,
           Wg1, bg1, Wg2, bg2, Wg3, bg3):
    nodes = _tc_nodes(x, W_en, b_en.reshape(1, 128))

    sr_idx = jnp.concatenate([senders, receivers])
    sr = _sc_gather(nodes, sr_idx)

    en = _tc_edge(sr, edge_attr, W_ee, b_ee.reshape(1, 128), We1[:128],
                  We1[128:256], We1[256:384], We2, be1.reshape(1, H1),
                  be2.reshape(1, 128))

    aggs = _sc_scatter(en, sr_idx)

    return _tc_node_global(
        nodes, aggs[0], aggs[1],
        Wn1[:128], Wn1[128:256], Wn1[256:384], bn1.reshape(1, H1),
        Wn2, bn2.reshape(1, 128),
        Wg1[:128], Wg1[128:256], bg1.reshape(1, H1),
        Wg2, bg2.reshape(1, 128), Wg3.reshape(1, 128), bg3.reshape(1, 1))


# speedup vs baseline: 2.5025x; 1.1409x over previous
import functools

import jax
import jax.numpy as jnp
from jax import lax
from jax.experimental import pallas as pl
from jax.experimental.pallas import tpu as pltpu
from jax.experimental.pallas import tpu_sc as plsc

N = 10000
E = 160000
H1 = 256

_NC = 2
_NS = 16
_NW = _NC * _NS

_C = 2
_EC = E // _C

f32 = jnp.float32


def _tc_nodes(x, W, b):
    def body(x_ref, w_ref, b_ref, o_ref):
        o_ref[...] = jnp.dot(x_ref[...], w_ref[...],
                             preferred_element_type=f32) + b_ref[...]

    return pl.pallas_call(
        body,
        grid=(10,),
        in_specs=[
            pl.BlockSpec((1000, 128), lambda i: (i, 0)),
            pl.BlockSpec((128, 128), lambda i: (0, 0)),
            pl.BlockSpec((1, 128), lambda i: (0, 0)),
        ],
        out_specs=pl.BlockSpec((1000, 128), lambda i: (i, 0)),
        out_shape=jax.ShapeDtypeStruct((N, 128), f32),
    )(x, W, b)


_G_CH = 128


def _sc_gather(table, idx, e0):
    per_w = _EC // _NS
    nfull = per_w // _G_CH
    npair = (nfull // 2) * 2
    tail = per_w - nfull * _G_CH
    mesh = plsc.VectorSubcoreMesh(core_axis_name="c", subcore_axis_name="s")

    @functools.partial(
        pl.kernel,
        out_type=jax.ShapeDtypeStruct((2 * _EC, 128), f32),
        mesh=mesh,
        scratch_types=[
            pltpu.VMEM((per_w,), jnp.int32),
            pltpu.VMEM((_G_CH, 128), f32),
            pltpu.VMEM((_G_CH, 128), f32),
            pltpu.SemaphoreType.DMA,
            pltpu.SemaphoreType.DMA,
        ],
    )
    def k(table_hbm, idx_hbm, out_hbm, idx_v, rows0, rows1, sem0, sem1):
        half = lax.axis_index("c")
        s = lax.axis_index("s")
        ibase = half * E + e0 + s * per_w
        obase = half * _EC + s * per_w
        pltpu.sync_copy(idx_hbm.at[pl.ds(ibase, per_w)], idx_v)

        rows = (rows0, rows1)
        sems = (sem0, sem1)

        def start(jj, b):
            pltpu.async_copy(
                table_hbm.at[idx_v.at[pl.ds(jj * _G_CH, _G_CH)]],
                rows[b], sems[b])

        def finish(b):
            pltpu.make_async_copy(
                table_hbm.at[idx_v.at[pl.ds(0, _G_CH)]],
                rows[b], sems[b]).wait()

        def store(jj, b):
            pltpu.sync_copy(rows[b],
                            out_hbm.at[pl.ds(obase + jj * _G_CH, _G_CH)])

        start(0, 0)

        @pl.loop(0, npair, step=2)
        def _(j):
            finish(0)
            start(j + 1, 1)
            store(j, 0)
            finish(1)

            @pl.when(j + 2 < nfull)
            def _():
                start(j + 2, 0)

            store(j + 1, 1)

        if nfull > npair:
            finish(0)
            store(nfull - 1, 0)

        if tail:
            pltpu.async_copy(
                table_hbm.at[idx_v.at[pl.ds(nfull * _G_CH, tail)]],
                rows1.at[pl.ds(0, tail)], sem1).wait()
            pltpu.sync_copy(rows1.at[pl.ds(0, tail)],
                            out_hbm.at[pl.ds(obase + nfull * _G_CH, tail)])

    return k(table, idx)


_BE = 2000


def _tc_edge(sr, ea, W_ee, b_ee, We1e, Ws, Wr, We2, be1, be2, c0):
    def body(snt_ref, rcv_ref, ea_ref, wee_ref, bee_ref, we1e_ref, ws_ref,
             wr_ref, we2_ref, be1_ref, be2_ref, o_ref):
        el = jnp.dot(ea_ref[...], wee_ref[...],
                     preferred_element_type=f32) + bee_ref[...]
        h = jnp.dot(el, we1e_ref[...], preferred_element_type=f32)
        h = h + jnp.dot(snt_ref[...], ws_ref[...], preferred_element_type=f32)
        h = h + jnp.dot(rcv_ref[...], wr_ref[...], preferred_element_type=f32)
        h = jnp.maximum(h + be1_ref[...], 0.0)
        o_ref[...] = jnp.dot(h, we2_ref[...],
                             preferred_element_type=f32) + be2_ref[...]

    nblk = _EC // _BE
    eoff = c0 // _BE
    return pl.pallas_call(
        body,
        grid=(nblk,),
        in_specs=[
            pl.BlockSpec((_BE, 128), lambda i: (i, 0)),
            pl.BlockSpec((_BE, 128), lambda i: (i + nblk, 0)),
            pl.BlockSpec((_BE, 16), lambda i: (i + eoff, 0)),
            pl.BlockSpec((16, 128), lambda i: (0, 0)),
            pl.BlockSpec((1, 128), lambda i: (0, 0)),
            pl.BlockSpec((128, 256), lambda i: (0, 0)),
            pl.BlockSpec((128, 256), lambda i: (0, 0)),
            pl.BlockSpec((128, 256), lambda i: (0, 0)),
            pl.BlockSpec((256, 128), lambda i: (0, 0)),
            pl.BlockSpec((1, 256), lambda i: (0, 0)),
            pl.BlockSpec((1, 128), lambda i: (0, 0)),
        ],
        out_specs=pl.BlockSpec((_BE, 128), lambda i: (i, 0)),
        out_shape=jax.ShapeDtypeStruct((_EC, 128), f32),
    )(sr, sr, ea, W_ee, b_ee, We1e, Ws, Wr, We2, be1, be2)


_S_CH = 64
_S_OCH = 200
_S_ONCH = N // _S_OCH
_S_ZCH = 40
_S_ZNCH = N // _S_ZCH


def _sc_scatter(en, sr_idx, e0):
    per_t = _EC // _NS
    nfull = per_t // _S_CH
    npair = (nfull // 2) * 2
    tail = per_t - nfull * _S_CH
    mesh = plsc.VectorSubcoreMesh(core_axis_name="c", subcore_axis_name="s")

    @functools.partial(
        pl.kernel,
        out_type=jax.ShapeDtypeStruct((2, N, 128), f32),
        mesh=mesh,
        scratch_types=[
            pltpu.VMEM_SHARED((N, 128), f32),
            pltpu.VMEM((_S_CH, 128), f32),
            pltpu.VMEM((_S_CH, 128), f32),
            pltpu.VMEM((per_t,), jnp.int32),
            pltpu.VMEM((_S_CH,), jnp.int32),
            pltpu.VMEM((_S_CH,), jnp.int32),
            pltpu.VMEM((max(tail, 1),), jnp.int32),
            pltpu.SemaphoreType.DMA,
            pltpu.SemaphoreType.DMA,
        ],
    )
    def k(en_hbm, sr_hbm, out_hbm, acc_sh, rows0, rows1, slab_v,
          idx0, idx1, idxt_v, sem0, sem1):
        c = lax.axis_index("c")
        s = lax.axis_index("s")

        @pl.loop(0, _S_ZCH)
        def _(i):
            for jj in range(8):
                rows0[i, pl.ds(jj * 16, 16)] = jnp.zeros((16,), f32)

        nzch = jnp.where(s < _S_ZNCH - (_S_ZNCH // _NS) * _NS,
                         _S_ZNCH // _NS + 1, _S_ZNCH // _NS)

        @pl.loop(0, nzch)
        def _(t):
            ch = s + t * _NS
            pltpu.sync_copy(rows0.at[pl.ds(0, _S_ZCH)],
                            acc_sh.at[pl.ds(ch * _S_ZCH, _S_ZCH)])

        plsc.subcore_barrier()

        ebase = s * per_t
        pltpu.sync_copy(sr_hbm.at[pl.ds(c * E + e0 + ebase, per_t)], slab_v)

        rows = (rows0, rows1)
        idxc = (idx0, idx1)
        sems = (sem0, sem1)

        def load(jj, b):
            pltpu.async_copy(en_hbm.at[pl.ds(ebase + jj * _S_CH, _S_CH)],
                             rows[b], sems[b])

        def loaded(b):
            pltpu.make_async_copy(en_hbm.at[pl.ds(ebase, _S_CH)],
                                  rows[b], sems[b]).wait()

        def scat(jj, b):
            for v in range(_S_CH // 16):
                idxc[b][pl.ds(v * 16, 16)] = (
                    slab_v[pl.ds(jj * _S_CH + v * 16, 16)])
            pltpu.sync_copy(rows[b], acc_sh.at[idxc[b]], add=True)

        load(0, 0)

        @pl.loop(0, npair, step=2)
        def _(j):
            loaded(0)
            load(j + 1, 1)
            scat(j, 0)
            loaded(1)

            @pl.when(j + 2 < nfull)
            def _():
                load(j + 2, 0)

            scat(j + 1, 1)

        if nfull > npair:
            loaded(0)
            scat(nfull - 1, 0)

        if tail:
            toff = ebase + nfull * _S_CH
            pltpu.sync_copy(en_hbm.at[pl.ds(toff, tail)],
                            rows1.at[pl.ds(0, tail)])
            pltpu.sync_copy(sr_hbm.at[pl.ds(c * E + e0 + toff, tail)], idxt_v)
            pltpu.sync_copy(rows1.at[pl.ds(0, tail)], acc_sh.at[idxt_v],
                            add=True)

        plsc.subcore_barrier()

        noch = jnp.where(s < _S_ONCH - (_S_ONCH // _NS) * _NS,
                         _S_ONCH // _NS + 1, _S_ONCH // _NS)

        @pl.loop(0, noch)
        def _(t):
            rr = (s + t * _NS) * _S_OCH
            pltpu.sync_copy(acc_sh.at[pl.ds(rr, _S_OCH)],
                            out_hbm.at[c, pl.ds(rr, _S_OCH)])

    return k(en, sr_idx)


_BN = 1000


def _tc_node_global(nodes, sents, recvs, Wn1n, Wn1s, Wn1r, bn1, Wn2, bn2,
                    Wg1n, Wg1e, bg1, Wg2, bg2, wg3row, bg3):
    G = N // _BN
    nagg = len(sents)

    def body(nd, *rest):
        agg_refs = rest[:2 * nagg]
        (wn1n, wn1s, wn1r, bn1r, wn2, bn2r, wg1n, wg1e, bg1r, wg2, bg2r,
         wg3r, bg3r, o_ref, acc_n, acc_e) = rest[2 * nagg:]
        i = pl.program_id(0)

        @pl.when(i == 0)
        def _():
            acc_n[...] = jnp.zeros_like(acc_n)
            acc_e[...] = jnp.zeros_like(acc_e)

        sa = agg_refs[0][...]
        ra = agg_refs[nagg][...]
        for t in range(1, nagg):
            sa = sa + agg_refs[t][...]
            ra = ra + agg_refs[nagg + t][...]

        h = jnp.dot(nd[...], wn1n[...], preferred_element_type=f32)
        h = h + jnp.dot(sa, wn1s[...], preferred_element_type=f32)
        h = h + jnp.dot(ra, wn1r[...], preferred_element_type=f32)
        h = jnp.maximum(h + bn1r[...], 0.0)
        nn = jnp.dot(h, wn2[...], preferred_element_type=f32) + bn2r[...]
        acc_n[...] += nn.reshape(_BN // 8, 8, 128).sum(0)
        acc_e[...] += sa.reshape(_BN // 8, 8, 128).sum(0)

        @pl.when(i == G - 1)
        def _():
            na = jnp.sum(acc_n[...], axis=0, keepdims=True)
            eg = jnp.sum(acc_e[...], axis=0, keepdims=True)
            hg = jnp.dot(na, wg1n[...], preferred_element_type=f32)
            hg = hg + jnp.dot(eg, wg1e[...], preferred_element_type=f32)
            hg = jnp.maximum(hg + bg1r[...], 0.0)
            hg2 = jnp.maximum(
                jnp.dot(hg, wg2[...], preferred_element_type=f32) + bg2r[...],
                0.0)
            o_ref[...] = (jnp.sum(hg2 * wg3r[...], axis=1, keepdims=True)
                          + bg3r[...])

    blk = pl.BlockSpec((_BN, 128), lambda i: (i, 0))
    wfull = lambda a, b: pl.BlockSpec((a, b), lambda i: (0, 0))

    return pl.pallas_call(
        body,
        grid=(G,),
        in_specs=([blk] * (1 + 2 * nagg) + [
            wfull(128, 256), wfull(128, 256), wfull(128, 256),
            wfull(1, 256), wfull(256, 128), wfull(1, 128),
            wfull(128, 256), wfull(128, 256), wfull(1, 256),
            wfull(256, 128), wfull(1, 128), wfull(1, 128), wfull(1, 1),
        ]),
        out_specs=pl.BlockSpec((1, 1), lambda i: (0, 0)),
        out_shape=jax.ShapeDtypeStruct((1, 1), f32),
        scratch_shapes=[pltpu.VMEM((8, 128), f32), pltpu.VMEM((8, 128), f32)],
    )(nodes, *sents, *recvs, Wn1n, Wn1s, Wn1r, bn1, Wn2, bn2,
      Wg1n, Wg1e, bg1, Wg2, bg2, wg3row, bg3)


def kernel(x, edge_attr, senders, receivers, W_en, b_en, W_ee, b_ee,
           We1, be1, We2, be2, Wn1, bn1, Wn2, bn2,
           Wg1, bg1, Wg2, bg2, Wg3, bg3):
    nodes = _tc_nodes(x, W_en, b_en.reshape(1, 128))

    sr_idx = jnp.concatenate([senders, receivers])

    sents, recvs = [], []
    for k in range(_C):
        e0 = k * _EC
        sr_k = _sc_gather(nodes, sr_idx, e0)
        en_k = _tc_edge(sr_k, edge_attr, W_ee, b_ee.reshape(1, 128),
                        We1[:128], We1[128:256], We1[256:384], We2,
                        be1.reshape(1, H1), be2.reshape(1, 128), e0)
        aggs_k = _sc_scatter(en_k, sr_idx, e0)
        sents.append(aggs_k[0])
        recvs.append(aggs_k[1])

    return _tc_node_global(
        nodes, sents, recvs,
        Wn1[:128], Wn1[128:256], Wn1[256:384], bn1.reshape(1, H1),
        Wn2, bn2.reshape(1, 128),
        Wg1[:128], Wg1[128:256], bg1.reshape(1, H1),
        Wg2, bg2.reshape(1, 128), Wg3.reshape(1, 128), bg3.reshape(1, 1))
